# baseline (speedup 1.0000x reference)
import functools

import jax
import jax.numpy as jnp
from jax import lax
from jax.experimental import pallas as pl
from jax.experimental.pallas import tpu as pltpu

_B, _N, _D = 4, 4096, 2048
_R = 64
_G = _B * _N // _R


@jax.jit
def _tc_gather(x_flat, idx):
    def body(idx_ref, x_hbm, out_ref, sem):
        g = pl.program_id(0)
        for r in range(_R):
            pltpu.make_async_copy(
                x_hbm.at[pl.ds(idx_ref[g * _R + r], 1)],
                out_ref.at[pl.ds(r, 1)],
                sem,
            ).start()
        for r in range(_R):
            pltpu.make_async_copy(
                x_hbm.at[pl.ds(0, 1)],
                out_ref.at[pl.ds(r, 1)],
                sem,
            ).wait()

    grid_spec = pltpu.PrefetchScalarGridSpec(
        num_scalar_prefetch=1,
        grid=(_G,),
        in_specs=[pl.BlockSpec(memory_space=pltpu.MemorySpace.HBM)],
        out_specs=pl.BlockSpec((_R, _D), lambda g, idx: (g, 0)),
        scratch_shapes=[pltpu.SemaphoreType.DMA],
    )
    return pl.pallas_call(
        body,
        grid_spec=grid_spec,
        out_shape=jax.ShapeDtypeStruct((_B * _N, _D), jnp.float32),
    )(idx, x_flat)


def _perm_indices(B, N):
    base_key = jax.random.key(42)

    def one(i):
        return jax.random.permutation(jax.random.fold_in(base_key, i), N)

    perm = jax.vmap(one)(jnp.arange(B))
    flat = perm.astype(jnp.int32) + (jnp.arange(B, dtype=jnp.int32) * N)[:, None]
    return flat.reshape(-1)


def kernel(x):
    B, N, D = x.shape
    idx = _perm_indices(B, N)
    out = _tc_gather(x.reshape(B * N, D), idx)
    return out.reshape(B, N, D)
